# baseline (speedup 1.0000x reference)
import functools

import jax
import jax.numpy as jnp
from jax import lax
from jax.experimental import pallas as pl
from jax.experimental.pallas import tpu as pltpu
from jax.experimental.pallas import tpu_sc as plsc

B, N, K = 16384, 1001, 10
QCAP = 1032
NSLAB = 8

NC, NS, L = 2, 16, 16
NW = NC * NS
ROWS_W = B // NW
GROUPS_W = ROWS_W // L
NGROUPS = B // L


def _insert(cand, y):
    new = []
    for k in range(K):
        ck = cand[k]
        new.append(jnp.maximum(ck, y))
        y = jnp.minimum(ck, y)
    return new


def _sc_body(scores_hbm, tail_hbm, cand_hbm, cnt_hbm, pos_hbm, *refs):
    bufs0 = refs[0:NSLAB]
    bufs1 = refs[NSLAB:2 * NSLAB]
    queue, cand_v, cnt_v, pos_v, sem0, sem1 = refs[2 * NSLAB:]

    wid = lax.axis_index("s") * NC + lax.axis_index("c")
    iota16 = lax.iota(jnp.int32, L)
    neg_inf = jnp.full((L,), -jnp.inf, jnp.float32)
    zeros = jnp.zeros((L,), jnp.float32)
    skew = iota16 * 8
    qbase = iota16 * QCAP

    for k in range(K, 16):
        cand_v[pl.ds(k * L, L)] = zeros

    def slab_pair(g, s, bufs):
        base = (wid * GROUPS_W + g) * L
        if s < NSLAB - 1:
            src = scores_hbm.at[pl.ds(base, L), pl.ds(s * 128, 128)]
        else:
            src = tail_hbm.at[pl.ds(base, L)]
        return src, bufs[s]

    def issue(g, bufs, sem):
        for s in range(NSLAB):
            src, dst = slab_pair(g, s, bufs)
            pltpu.make_async_copy(src, dst, sem).start()

    def drain_dma(g, bufs, sem):
        for s in range(NSLAB):
            src, dst = slab_pair(g, s, bufs)
            pltpu.make_async_copy(src, dst, sem).wait()

    UNR = 16

    def process(g, bufs):
        gi = wid * GROUPS_W + g
        pos = plsc.load_gather(bufs[0], [iota16, jnp.zeros((L,), jnp.int32)])

        def warm_body(tc, c):
            cand, cn = list(c[:K]), c[K]
            for i in range(UNR):
                col = (tc * UNR + i + skew) & 127
                x = plsc.load_gather(bufs[0], [iota16, col])
                cand = _insert(cand, x)
                cn = cn + (x > pos).astype(jnp.float32)
            return tuple(cand) + (cn,)

        res = lax.fori_loop(0, 128 // UNR, warm_body,
                            tuple([neg_inf] * K) + (zeros,))
        cand, cnt = list(res[:K]), res[K]
        thr = cand[K - 1]

        qptr = jnp.zeros((L,), jnp.int32)
        for s in range(1, NSLAB):
            def app_body(tc, c, s=s):
                qp, cn = c
                xs, ms = [], []
                for i in range(UNR):
                    col = (tc * UNR + i + skew) & 127
                    x = plsc.load_gather(bufs[s], [iota16, col])
                    hit = x > pos
                    m = x >= thr
                    if s == NSLAB - 1:
                        valid = col < (N - 896)
                        hit = hit & valid
                        m = m & valid
                    cn = cn + hit.astype(jnp.float32)
                    xs.append(x)
                    ms.append(m)
                offs = [qp]
                for i in range(UNR - 1):
                    offs.append(offs[-1] + ms[i].astype(jnp.int32))
                for i in range(UNR):
                    plsc.store_scatter(queue, [qbase + offs[i]], xs[i],
                                       mask=ms[i])
                return (offs[-1] + ms[-1].astype(jnp.int32), cn)

            qptr, cnt = lax.fori_loop(0, 128 // UNR, app_body, (qptr, cnt))

        maxlen = lax.reduce_max(qptr, (0,))

        def drain_body(tc, c):
            cand = list(c)
            for i in range(UNR):
                idx = tc * UNR + i
                x = plsc.load_gather(queue, [qbase + idx])
                x = jnp.where(idx < qptr, x, neg_inf)
                cand = _insert(cand, x)
            return tuple(cand)

        nch = lax.div(maxlen + (UNR - 1), UNR)
        cand = lax.fori_loop(0, nch, drain_body, tuple(cand))
        for k in range(K):
            cand_v[pl.ds(k * L, L)] = cand[k]
        cnt_v[...] = cnt
        pos_v[...] = pos
        pltpu.sync_copy(cand_v, cand_hbm.at[gi])
        pltpu.sync_copy(cnt_v, cnt_hbm.at[gi])
        pltpu.sync_copy(pos_v, pos_hbm.at[gi])

    issue(0, bufs0, sem0)

    def group_body(g, carry):
        def step(bufs_a, sem_a, bufs_b, sem_b):
            @pl.when(g + 1 < GROUPS_W)
            def _():
                issue(g + 1, bufs_b, sem_b)

            drain_dma(g, bufs_a, sem_a)
            process(g, bufs_a)

        @pl.when(g % 2 == 0)
        def _():
            step(bufs0, sem0, bufs1, sem1)

        @pl.when(g % 2 == 1)
        def _():
            step(bufs1, sem1, bufs0, sem0)

        return carry

    lax.fori_loop(0, GROUPS_W, group_body, 0)


G2 = 128


def _loss_body(cand_ref, cnt_ref, pos_ref, num_ref, den_ref):
    i = pl.program_id(0)
    v = cand_ref[...]
    p = cnt_ref[...]
    pos = pos_ref[...]
    r = lax.broadcasted_iota(jnp.int32, (G2, 16, 16), 1).astype(jnp.float32)
    p3 = p[:, None, :]
    pos3 = pos[:, None, :]
    w = 1.0 / jnp.log2(r + 2.0)
    bpr = -jnp.log(jnp.clip(jax.nn.sigmoid(pos3 - v), 1e-8))
    valid = (r < float(K)) & (r != p3)
    num = jnp.sum(jnp.where(valid, bpr * w, 0.0))
    den = jnp.sum(10.0 - (p < float(K)).astype(jnp.float32))

    @pl.when(i == 0)
    def _():
        num_ref[...] = jnp.zeros((1, 1), jnp.float32)
        den_ref[...] = jnp.zeros((1, 1), jnp.float32)

    num_ref[...] += num.reshape(1, 1)
    den_ref[...] += den.reshape(1, 1)


def kernel(scores):
    mesh = plsc.VectorSubcoreMesh(core_axis_name="c", subcore_axis_name="s")
    sc = pl.kernel(
        _sc_body,
        mesh=mesh,
        compiler_params=pltpu.CompilerParams(needs_layout_passes=False),
        out_type=[
            jax.ShapeDtypeStruct((NGROUPS, 16 * L), jnp.float32),
            jax.ShapeDtypeStruct((NGROUPS, L), jnp.float32),
            jax.ShapeDtypeStruct((NGROUPS, L), jnp.float32),
        ],
        scratch_types=(
            [pltpu.VMEM((L, 128), jnp.float32)] * (2 * NSLAB)
            + [
                pltpu.VMEM((L * QCAP,), jnp.float32),
                pltpu.VMEM((16 * L,), jnp.float32),
                pltpu.VMEM((L,), jnp.float32),
                pltpu.VMEM((L,), jnp.float32),
                pltpu.SemaphoreType.DMA,
                pltpu.SemaphoreType.DMA,
            ]
        ),
    )
    tail = jnp.pad(scores[:, 896:], ((0, 0), (0, 128 - (N - 896))))
    cand, cnt, posv = sc(scores, tail)
    cand = cand.reshape(NGROUPS, 16, L)

    num, den = pl.pallas_call(
        _loss_body,
        grid=(NGROUPS // G2,),
        in_specs=[
            pl.BlockSpec((G2, 16, L), lambda i: (i, 0, 0)),
            pl.BlockSpec((G2, L), lambda i: (i, 0)),
            pl.BlockSpec((G2, L), lambda i: (i, 0)),
        ],
        out_specs=[
            pl.BlockSpec((1, 1), lambda i: (0, 0)),
            pl.BlockSpec((1, 1), lambda i: (0, 0)),
        ],
        out_shape=[
            jax.ShapeDtypeStruct((1, 1), jnp.float32),
            jax.ShapeDtypeStruct((1, 1), jnp.float32),
        ],
    )(cand, cnt, posv)
    return num[0, 0] / jnp.clip(den[0, 0], 1.0)


# speedup vs baseline: 1.1746x; 1.1746x over previous
import functools

import jax
import jax.numpy as jnp
from jax import lax
from jax.experimental import pallas as pl
from jax.experimental.pallas import tpu as pltpu
from jax.experimental.pallas import tpu_sc as plsc

B, N, K = 16384, 1001, 10
QCAP = 1032
NSLAB = 8

NC, NS, L = 2, 16, 16
NW = NC * NS
ROWS_W = B // NW
GROUPS_W = ROWS_W // L
NGROUPS = B // L


def _insert(cand, y):
    new = []
    for k in range(K):
        ck = cand[k]
        new.append(jnp.maximum(ck, y))
        y = jnp.minimum(ck, y)
    return new


def _sc_body(scores_hbm, tail_hbm, cand_hbm, pos_hbm, *refs):
    bufs0 = refs[0:NSLAB]
    bufs1 = refs[NSLAB:2 * NSLAB]
    queue, cand_v, pos_v, sem0, sem1 = refs[2 * NSLAB:]

    wid = lax.axis_index("s") * NC + lax.axis_index("c")
    iota16 = lax.iota(jnp.int32, L)
    neg_inf = jnp.full((L,), -jnp.inf, jnp.float32)
    zeros = jnp.zeros((L,), jnp.float32)
    skew = iota16 * 8
    qbase = iota16 * QCAP

    for k in range(K, 16):
        cand_v[pl.ds(k * L, L)] = zeros

    def slab_pair(g, s, bufs):
        base = (wid * GROUPS_W + g) * L
        if s < NSLAB - 1:
            src = scores_hbm.at[pl.ds(base, L), pl.ds(s * 128, 128)]
        else:
            src = tail_hbm.at[pl.ds(base, L)]
        return src, bufs[s]

    def issue(g, bufs, sem):
        for s in range(NSLAB):
            src, dst = slab_pair(g, s, bufs)
            pltpu.make_async_copy(src, dst, sem).start()

    def drain_dma(g, bufs, sem):
        for s in range(NSLAB):
            src, dst = slab_pair(g, s, bufs)
            pltpu.make_async_copy(src, dst, sem).wait()

    UNR = 8

    def process(g, bufs):
        gi = wid * GROUPS_W + g
        pos = plsc.load_gather(bufs[0], [iota16, jnp.zeros((L,), jnp.int32)])

        def warm_body(tc, c):
            cand = list(c)
            for i in range(UNR):
                col = (tc * UNR + i + skew) & 127
                x = plsc.load_gather(bufs[0], [iota16, col])
                cand = _insert(cand, x)
            return tuple(cand)

        res = lax.fori_loop(0, 128 // UNR, warm_body, tuple([neg_inf] * K))
        cand = list(res)
        thr = cand[K - 1]

        qptr = jnp.zeros((L,), jnp.int32)
        for s in range(1, NSLAB):
            def app_body(tc, qp, s=s):
                xs, ms = [], []
                for i in range(UNR):
                    col = (tc * UNR + i + skew) & 127
                    x = plsc.load_gather(bufs[s], [iota16, col])
                    m = x >= thr
                    if s == NSLAB - 1:
                        m = m & (col < (N - 896))
                    xs.append(x)
                    ms.append(m)
                offs = [qp]
                for i in range(UNR - 1):
                    offs.append(offs[-1] + ms[i].astype(jnp.int32))
                for i in range(UNR):
                    plsc.store_scatter(queue, [qbase + offs[i]], xs[i],
                                       mask=ms[i])
                return offs[-1] + ms[-1].astype(jnp.int32)

            qptr = lax.fori_loop(0, 128 // UNR, app_body, qptr)

        maxlen = lax.reduce_max(qptr, (0,))

        def drain_body(tc, c):
            cand = list(c)
            for i in range(UNR):
                idx = tc * UNR + i
                x = plsc.load_gather(queue, [qbase + idx])
                x = jnp.where(idx < qptr, x, neg_inf)
                cand = _insert(cand, x)
            return tuple(cand)

        nch = lax.div(maxlen + (UNR - 1), UNR)
        cand = lax.fori_loop(0, nch, drain_body, tuple(cand))
        for k in range(K):
            cand_v[pl.ds(k * L, L)] = cand[k]
        pos_v[...] = pos
        pltpu.sync_copy(cand_v, cand_hbm.at[gi])
        pltpu.sync_copy(pos_v, pos_hbm.at[gi])

    issue(0, bufs0, sem0)

    def group_body(g, carry):
        def step(bufs_a, sem_a, bufs_b, sem_b):
            @pl.when(g + 1 < GROUPS_W)
            def _():
                issue(g + 1, bufs_b, sem_b)

            drain_dma(g, bufs_a, sem_a)
            process(g, bufs_a)

        @pl.when(g % 2 == 0)
        def _():
            step(bufs0, sem0, bufs1, sem1)

        @pl.when(g % 2 == 1)
        def _():
            step(bufs1, sem1, bufs0, sem0)

        return carry

    lax.fori_loop(0, GROUPS_W, group_body, 0)


G2 = 128


def _loss_body(cand_ref, pos_ref, num_ref, den_ref):
    i = pl.program_id(0)
    v = cand_ref[...]
    pos = pos_ref[...]
    r = lax.broadcasted_iota(jnp.int32, (G2, 16, 16), 1).astype(jnp.float32)
    pos3 = pos[:, None, :]
    above = jnp.where(r < float(K), (v > pos3).astype(jnp.float32), 0.0)
    p3 = jnp.sum(above, axis=1, keepdims=True)
    w = 1.0 / jnp.log2(r + 2.0)
    bpr = -jnp.log(jnp.clip(jax.nn.sigmoid(pos3 - v), 1e-8))
    valid = (r < float(K)) & (r != p3)
    num = jnp.sum(jnp.where(valid, bpr * w, 0.0))
    den = jnp.sum(10.0 - (p3 < float(K)).astype(jnp.float32))

    @pl.when(i == 0)
    def _():
        num_ref[...] = jnp.zeros((1, 1), jnp.float32)
        den_ref[...] = jnp.zeros((1, 1), jnp.float32)

    num_ref[...] += num.reshape(1, 1)
    den_ref[...] += den.reshape(1, 1)


def kernel(scores):
    mesh = plsc.VectorSubcoreMesh(core_axis_name="c", subcore_axis_name="s")
    sc = pl.kernel(
        _sc_body,
        mesh=mesh,
        compiler_params=pltpu.CompilerParams(needs_layout_passes=False),
        out_type=[
            jax.ShapeDtypeStruct((NGROUPS, 16 * L), jnp.float32),
            jax.ShapeDtypeStruct((NGROUPS, L), jnp.float32),
        ],
        scratch_types=(
            [pltpu.VMEM((L, 128), jnp.float32)] * (2 * NSLAB)
            + [
                pltpu.VMEM((L * QCAP,), jnp.float32),
                pltpu.VMEM((16 * L,), jnp.float32),
                pltpu.VMEM((L,), jnp.float32),
                pltpu.SemaphoreType.DMA,
                pltpu.SemaphoreType.DMA,
            ]
        ),
    )
    tail = jnp.pad(scores[:, 896:], ((0, 0), (0, 128 - (N - 896))))
    cand, posv = sc(scores, tail)
    cand = cand.reshape(NGROUPS, 16, L)

    num, den = pl.pallas_call(
        _loss_body,
        grid=(NGROUPS // G2,),
        in_specs=[
            pl.BlockSpec((G2, 16, L), lambda i: (i, 0, 0)),
            pl.BlockSpec((G2, L), lambda i: (i, 0)),
        ],
        out_specs=[
            pl.BlockSpec((1, 1), lambda i: (0, 0)),
            pl.BlockSpec((1, 1), lambda i: (0, 0)),
        ],
        out_shape=[
            jax.ShapeDtypeStruct((1, 1), jnp.float32),
            jax.ShapeDtypeStruct((1, 1), jnp.float32),
        ],
    )(cand, posv)
    return num[0, 0] / jnp.clip(den[0, 0], 1.0)
